# baseline (speedup 1.0000x reference)
import functools

import jax
import jax.numpy as jnp
from jax import lax
from jax.experimental import pallas as pl
from jax.experimental.pallas import tpu as pltpu
from jax.experimental.pallas import tpu_sc as plsc

N_NODES = 10000
N_EDGES = 320000
EMB = 128

NC = 2
NS = 16
NW = NC * NS
C = 40
K = 2
NCH = N_EDGES // C
TCH = NCH // NW
ITERS = TCH // K
ZROWS = 40
NODE_CHUNKS = N_NODES // ZROWS
RR = -(-NODE_CHUNKS // NS)


@functools.partial(
    pl.kernel,
    mesh=plsc.VectorSubcoreMesh(core_axis_name="c", subcore_axis_name="s"),
    out_type=jax.ShapeDtypeStruct((NC, N_NODES, EMB), jnp.float32),
    scratch_types=[
        pltpu.VMEM((K, 2, C), jnp.int32),
        pltpu.VMEM((K, C, EMB), jnp.float32),
        pltpu.VMEM((K, C, EMB), jnp.float32),
        pltpu.VMEM_SHARED((N_NODES, EMB), jnp.float32),
        pltpu.SemaphoreType.DMA,
        pltpu.SemaphoreType.DMA,
        pltpu.SemaphoreType.DMA,
        pltpu.SemaphoreType.DMA,
        pltpu.SemaphoreType.DMA,
        pltpu.SemaphoreType.DMA,
    ],
)
def _edge_agg(x_hbm, idx_hbm, ea_hbm, out_hbm,
              idx_v, gbuf, ebuf, agg_sh,
              g0, g1, e0, e1, s0, s1):
    gsems = (g0, g1)
    esems = (e0, e1)
    ssems = (s0, s1)
    c = lax.axis_index("c")
    s = lax.axis_index("s")
    w = c * NS + s
    qbase = w * TCH

    def _zrow(i, _):
        for k in range(EMB // 16):
            ebuf[0, i, pl.ds(k * 16, 16)] = jnp.zeros((16,), jnp.float32)
        return 0
    lax.fori_loop(0, ZROWS, _zrow, 0)

    def _zero(i, _):
        j = s + i * NS

        @pl.when(j < NODE_CHUNKS)
        def _():
            pltpu.sync_copy(ebuf.at[0], agg_sh.at[pl.ds(j * ZROWS, ZROWS)])
        return 0
    lax.fori_loop(0, RR, _zero, 0)
    plsc.subcore_barrier()

    def _iter(i, _):
        q0 = qbase + i * K
        pltpu.sync_copy(idx_hbm.at[pl.ds(q0, K)], idx_v)

        gd = [pltpu.async_copy(x_hbm.at[idx_v.at[k, 0]],
                               gbuf.at[k], gsems[k])
              for k in range(K)]
        ed = [pltpu.async_copy(ea_hbm.at[q0 + k],
                               ebuf.at[k], esems[k])
              for k in range(K)]

        sd = []
        for k in range(K):
            gd[k].wait()
            ed[k].wait()

            def _row(r, _, k=k):
                for t in range(EMB // 16):
                    v = gbuf[k, r, pl.ds(t * 16, 16)] \
                        + ebuf[k, r, pl.ds(t * 16, 16)]
                    gbuf[k, r, pl.ds(t * 16, 16)] = jnp.maximum(v, 0.0)
                return 0
            lax.fori_loop(0, C, _row, 0)

            sd.append(pltpu.async_copy(
                gbuf.at[k], agg_sh.at[idx_v.at[k, 1]], ssems[k], add=True))

        for d in sd:
            d.wait()
        return 0
    lax.fori_loop(0, ITERS, _iter, 0)

    plsc.subcore_barrier()

    def _out(i, _):
        j = s + i * NS

        @pl.when(j < NODE_CHUNKS)
        def _():
            b = j * ZROWS
            pltpu.sync_copy(agg_sh.at[pl.ds(b, ZROWS)], gbuf.at[0])
            pltpu.sync_copy(gbuf.at[0], out_hbm.at[c].at[pl.ds(b, ZROWS)])
        return 0
    lax.fori_loop(0, RR, _out, 0)


def _mlp_body(x_ref, a0_ref, a1_ref, w1_ref, b1_ref, w2_ref, b2_ref, o_ref):
    h = x_ref[...] + a0_ref[...] + a1_ref[...]
    h = jnp.dot(h, w1_ref[...], preferred_element_type=jnp.float32)
    h = jnp.maximum(h + b1_ref[...], 0.0)
    o_ref[...] = (
        jnp.dot(h, w2_ref[...], preferred_element_type=jnp.float32)
        + b2_ref[...]
    )


_ROW_BLK = 1000


def _mlp(x, a0, a1, W1, b1, W2, b2):
    return pl.pallas_call(
        _mlp_body,
        grid=(N_NODES // _ROW_BLK,),
        in_specs=[
            pl.BlockSpec((_ROW_BLK, EMB), lambda i: (i, 0)),
            pl.BlockSpec((_ROW_BLK, EMB), lambda i: (i, 0)),
            pl.BlockSpec((_ROW_BLK, EMB), lambda i: (i, 0)),
            pl.BlockSpec((EMB, 2 * EMB), lambda i: (0, 0)),
            pl.BlockSpec((1, 2 * EMB), lambda i: (0, 0)),
            pl.BlockSpec((2 * EMB, EMB), lambda i: (0, 0)),
            pl.BlockSpec((1, EMB), lambda i: (0, 0)),
        ],
        out_specs=pl.BlockSpec((_ROW_BLK, EMB), lambda i: (i, 0)),
        out_shape=jax.ShapeDtypeStruct((N_NODES, EMB), jnp.float32),
    )(x, a0, a1, W1, b1.reshape(1, -1), W2, b2.reshape(1, -1))


@jax.jit
def kernel(x, edge_index, edge_attr, W1, b1, W2, b2):
    ei = edge_index.astype(jnp.int32).reshape(2, NCH, C)
    idx = jnp.swapaxes(ei, 0, 1)
    ea = edge_attr.reshape(NCH, C, EMB)
    partials = _edge_agg(x, idx, ea)
    return _mlp(x, partials[0], partials[1], W1, b1, W2, b2)


# speedup vs baseline: 1.1229x; 1.1229x over previous
import functools

import jax
import jax.numpy as jnp
from jax import lax
from jax.experimental import pallas as pl
from jax.experimental.pallas import tpu as pltpu
from jax.experimental.pallas import tpu_sc as plsc

N_NODES = 10000
N_EDGES = 320000
EMB = 128

NC = 2
NS = 16
NW = NC * NS
C = 40
K = 2
NCH = N_EDGES // C
TCH = NCH // NW
ITERS = TCH // K
IBLK = 10
ZROWS = 40
NODE_CHUNKS = N_NODES // ZROWS
RR = -(-NODE_CHUNKS // NS)


@functools.partial(
    pl.kernel,
    mesh=plsc.VectorSubcoreMesh(core_axis_name="c", subcore_axis_name="s"),
    out_type=jax.ShapeDtypeStruct((NC, N_NODES, EMB), jnp.float32),
    scratch_types=[
        pltpu.VMEM((IBLK, 2, C), jnp.int32),
        pltpu.VMEM((K, C, EMB), jnp.float32),
        pltpu.VMEM((K, C, EMB), jnp.float32),
        pltpu.VMEM_SHARED((N_NODES, EMB), jnp.float32),
        pltpu.SemaphoreType.DMA,
        pltpu.SemaphoreType.DMA,
        pltpu.SemaphoreType.DMA,
        pltpu.SemaphoreType.DMA,
        pltpu.SemaphoreType.DMA,
        pltpu.SemaphoreType.DMA,
    ],
)
def _edge_agg(x_hbm, idx_hbm, ea_hbm, out_hbm,
              idx_v, gbuf, ebuf, agg_sh,
              g0, g1, e0, e1, s0, s1):
    gsems = (g0, g1)
    esems = (e0, e1)
    ssems = (s0, s1)
    c = lax.axis_index("c")
    s = lax.axis_index("s")
    w = c * NS + s
    qbase = w * TCH

    def _zrow(i, _):
        for k in range(EMB // 16):
            ebuf[0, i, pl.ds(k * 16, 16)] = jnp.zeros((16,), jnp.float32)
        return 0
    lax.fori_loop(0, ZROWS, _zrow, 0)

    def _zero(i, _):
        j = s + i * NS

        @pl.when(j < NODE_CHUNKS)
        def _():
            pltpu.sync_copy(ebuf.at[0], agg_sh.at[pl.ds(j * ZROWS, ZROWS)])
        return 0
    lax.fori_loop(0, RR, _zero, 0)
    plsc.subcore_barrier()

    def _iter(i, _):
        q0 = qbase + i * K
        local = lax.rem(i, IBLK // K) * K

        @pl.when(lax.rem(i, IBLK // K) == 0)
        def _():
            pltpu.sync_copy(idx_hbm.at[pl.ds(q0, IBLK)], idx_v)

        gd = [pltpu.async_copy(x_hbm.at[idx_v.at[local + k, 0]],
                               gbuf.at[k], gsems[k])
              for k in range(K)]
        ed = [pltpu.async_copy(ea_hbm.at[q0 + k],
                               ebuf.at[k], esems[k])
              for k in range(K)]

        sd = []
        for k in range(K):
            gd[k].wait()
            ed[k].wait()

            def _row(r, _, k=k):
                for t in range(EMB // 16):
                    v = gbuf[k, r, pl.ds(t * 16, 16)] \
                        + ebuf[k, r, pl.ds(t * 16, 16)]
                    gbuf[k, r, pl.ds(t * 16, 16)] = jnp.maximum(v, 0.0)
                return 0
            lax.fori_loop(0, C, _row, 0)

            sd.append(pltpu.async_copy(
                gbuf.at[k], agg_sh.at[idx_v.at[local + k, 1]],
                ssems[k], add=True))

        for d in sd:
            d.wait()
        return 0
    lax.fori_loop(0, ITERS, _iter, 0)

    plsc.subcore_barrier()

    def _out(i, _):
        j = s + i * NS

        @pl.when(j < NODE_CHUNKS)
        def _():
            b = j * ZROWS
            pltpu.sync_copy(agg_sh.at[pl.ds(b, ZROWS)], gbuf.at[0])
            pltpu.sync_copy(gbuf.at[0], out_hbm.at[c].at[pl.ds(b, ZROWS)])
        return 0
    lax.fori_loop(0, RR, _out, 0)


def _mlp_body(x_ref, a0_ref, a1_ref, w1_ref, b1_ref, w2_ref, b2_ref, o_ref):
    h = x_ref[...] + a0_ref[...] + a1_ref[...]
    h = jnp.dot(h, w1_ref[...], preferred_element_type=jnp.float32)
    h = jnp.maximum(h + b1_ref[...], 0.0)
    o_ref[...] = (
        jnp.dot(h, w2_ref[...], preferred_element_type=jnp.float32)
        + b2_ref[...]
    )


_ROW_BLK = 1000


def _mlp(x, a0, a1, W1, b1, W2, b2):
    return pl.pallas_call(
        _mlp_body,
        grid=(N_NODES // _ROW_BLK,),
        in_specs=[
            pl.BlockSpec((_ROW_BLK, EMB), lambda i: (i, 0)),
            pl.BlockSpec((_ROW_BLK, EMB), lambda i: (i, 0)),
            pl.BlockSpec((_ROW_BLK, EMB), lambda i: (i, 0)),
            pl.BlockSpec((EMB, 2 * EMB), lambda i: (0, 0)),
            pl.BlockSpec((1, 2 * EMB), lambda i: (0, 0)),
            pl.BlockSpec((2 * EMB, EMB), lambda i: (0, 0)),
            pl.BlockSpec((1, EMB), lambda i: (0, 0)),
        ],
        out_specs=pl.BlockSpec((_ROW_BLK, EMB), lambda i: (i, 0)),
        out_shape=jax.ShapeDtypeStruct((N_NODES, EMB), jnp.float32),
    )(x, a0, a1, W1, b1.reshape(1, -1), W2, b2.reshape(1, -1))


@jax.jit
def kernel(x, edge_index, edge_attr, W1, b1, W2, b2):
    ei = edge_index.astype(jnp.int32).reshape(2, NCH, C)
    idx = jnp.swapaxes(ei, 0, 1)
    ea = edge_attr.reshape(NCH, C, EMB)
    partials = _edge_agg(x, idx, ea)
    return _mlp(x, partials[0], partials[1], W1, b1, W2, b2)
